# baseline (speedup 1.0000x reference)
import jax
import jax.numpy as jnp
from jax import lax
from jax.experimental import pallas as pl
from jax.experimental.pallas import tpu as pltpu
from jax.experimental.pallas import tpu_sc as plsc

B, L, N, C = 32, 12, 2000, 4
DW = 32
STEP_PER_DAY = 23
DAY_PER_WEEK = 6

NW = 32
GROUPS = N // 16


def _sc_embed(x_hbm, wb_hbm, tod_hbm, dow_hbm, adpt_hbm, out_hbm,
              x_v0, x_v1, adp_sh, buf_a, buf_b, buf_c, wb_v, tod_v, dow_v,
              sem_x0, sem_x1, sem_st, sem_ao, sem_a, sem_b, sem_c):
    nc = 2
    sid = lax.axis_index("s")
    wid = sid * nc + lax.axis_index("c")

    pltpu.make_async_copy(x_hbm.at[wid, 0], x_v0, sem_x0).start()

    @pl.when(sid < L)
    def _():
        cp = pltpu.make_async_copy(
            adpt_hbm.at[pl.ds(sid * 8, 8)], adp_sh.at[pl.ds(sid * 8, 8)],
            sem_st)
        cp.start()
        cp.wait()

    pltpu.sync_copy(wb_hbm, wb_v)
    pltpu.sync_copy(tod_hbm, tod_v)
    pltpu.sync_copy(dow_hbm, dow_v)
    plsc.subcore_barrier()

    iota = lax.iota(jnp.int32, 16)
    zeros = jnp.zeros((16,), jnp.int32)
    w_spl = [plsc.load_gather(wb_v, [zeros + (8 + c)]) for c in range(8)]
    b_spl = [plsc.load_gather(wb_v, [zeros + (16 + c)]) for c in range(8)]

    def x_in(li, x_v, sem):
        return pltpu.make_async_copy(x_hbm.at[wid, li], x_v, sem)

    def adp_out(li):
        return pltpu.make_async_copy(
            adp_sh.at[pl.ds(li * 8, 8)],
            out_hbm.at[wid, li, pl.ds(24, 8)], sem_ao)

    def sec_out(buf, li, c0, sem):
        return pltpu.make_async_copy(
            buf, out_hbm.at[wid, li, pl.ds(c0, 8)], sem)

    def proj_sec(x_v, li):
        @plsc.parallel_loop(0, GROUPS, unroll=4)
        def proj_body(g):
            x0 = x_v[0, pl.ds(g * 16, 16)]
            for c in range(8):
                buf_a[c, pl.ds(g * 16, 16)] = x0 * w_spl[c] + b_spl[c]
        sec_out(buf_a, li, 0, sem_a).start()

    def tod_sec(x_v, li):
        @plsc.parallel_loop(0, GROUPS, unroll=4)
        def tod_body(g):
            x3 = x_v[3, pl.ds(g * 16, 16)]
            ti = (x3 * float(STEP_PER_DAY)).astype(jnp.int32)
            for c in range(8):
                buf_b[c, pl.ds(g * 16, 16)] = plsc.load_gather(
                    tod_v, [zeros + c, ti])
        sec_out(buf_b, li, 8, sem_b).start()

    def dow_sec(x_v, li):
        @plsc.parallel_loop(0, GROUPS, unroll=4)
        def dow_body(g):
            x2 = x_v[2, pl.ds(g * 16, 16)]
            di = (x2 * float(DAY_PER_WEEK)).astype(jnp.int32)
            for c in range(8):
                buf_c[c, pl.ds(g * 16, 16)] = plsc.load_gather(
                    dow_v, [zeros + c, di])
        sec_out(buf_c, li, 16, sem_c).start()

    def phase(li, x_v, sem_x):
        x_in(li, x_v, sem_x).wait()

        @pl.when(li > 0)
        def _():
            sec_out(buf_a, li, 0, sem_a).wait()
            sec_out(buf_b, li, 8, sem_b).wait()
            sec_out(buf_c, li, 16, sem_c).wait()
            adp_out(li).wait()

        proj_sec(x_v, li)
        tod_sec(x_v, li)
        dow_sec(x_v, li)
        adp_out(li).start()

    def l_body(i, carry):
        l0 = 2 * i
        l1 = l0 + 1
        phase(l0, x_v0, sem_x0)
        x_in(l1, x_v1, sem_x1).start()
        phase(l1, x_v1, sem_x1)

        @pl.when(i < (L // 2) - 1)
        def _():
            x_in(l1 + 1, x_v0, sem_x0).start()
        return carry

    lax.fori_loop(0, L // 2, l_body, 0)

    sec_out(buf_a, L - 1, 0, sem_a).wait()
    sec_out(buf_b, L - 1, 8, sem_b).wait()
    sec_out(buf_c, L - 1, 16, sem_c).wait()
    adp_out(L - 1).wait()


@jax.jit
def _run(x_t, wb, tod_t, dow_t, adp_t):
    fn = pl.kernel(
        _sc_embed,
        out_type=jax.ShapeDtypeStruct((B, L, DW, N), jnp.float32),
        mesh=plsc.VectorSubcoreMesh(core_axis_name="c", subcore_axis_name="s"),
        compiler_params=pltpu.CompilerParams(needs_layout_passes=False),
        scratch_types=[
            pltpu.VMEM((C, N), jnp.float32),
            pltpu.VMEM((C, N), jnp.float32),
            pltpu.VMEM_SHARED(((L + 1) * 8, N), jnp.float32),
            pltpu.VMEM((8, N), jnp.float32),
            pltpu.VMEM((8, N), jnp.float32),
            pltpu.VMEM((8, N), jnp.float32),
            pltpu.VMEM((24,), jnp.float32),
            pltpu.VMEM((8, 24), jnp.float32),
            pltpu.VMEM((8, 8), jnp.float32),
            pltpu.SemaphoreType.DMA,
            pltpu.SemaphoreType.DMA,
            pltpu.SemaphoreType.DMA,
            pltpu.SemaphoreType.DMA,
            pltpu.SemaphoreType.DMA,
            pltpu.SemaphoreType.DMA,
            pltpu.SemaphoreType.DMA,
        ],
    )
    return fn(x_t, wb, tod_t, dow_t, adp_t)


def kernel(x, W_in, b_in, tod_table, dow_table, adp):
    x_t = x.transpose(0, 1, 3, 2)
    adp_t = adp.transpose(0, 2, 1).reshape(L * 8, N)
    wb = jnp.concatenate([jnp.zeros((8,), jnp.float32), W_in[0], b_in])
    tod_t = tod_table.T
    dow_t = jnp.zeros((8, 8), jnp.float32).at[:, :7].set(dow_table.T)
    out_t = _run(x_t, wb, tod_t, dow_t, adp_t)
    return out_t.transpose(0, 1, 3, 2)


# speedup vs baseline: 1.0022x; 1.0022x over previous
import jax
import jax.numpy as jnp
from jax import lax
from jax.experimental import pallas as pl
from jax.experimental.pallas import tpu as pltpu
from jax.experimental.pallas import tpu_sc as plsc

B, L, N, C = 32, 12, 2000, 4
DW = 32
STEP_PER_DAY = 23
DAY_PER_WEEK = 6

NW = 32
GROUPS = N // 16


def _sc_embed(x_hbm, wb_hbm, tod_hbm, dow_hbm, adpt_hbm, out_hbm,
              x_v0, x_v1, adp_sh, buf_a, buf_b, buf_c, wb_v, tod_v, dow_v,
              sem_x0, sem_x1, sem_st, sem_ao, sem_a, sem_b, sem_c):
    nc = 2
    sid = lax.axis_index("s")
    wid = sid * nc + lax.axis_index("c")

    pltpu.make_async_copy(x_hbm.at[wid, 0], x_v0, sem_x0).start()

    @pl.when(sid < L)
    def _():
        cp = pltpu.make_async_copy(
            adpt_hbm.at[pl.ds(sid * 8, 8)], adp_sh.at[pl.ds(sid * 8, 8)],
            sem_st)
        cp.start()
        cp.wait()

    pltpu.sync_copy(wb_hbm, wb_v)
    pltpu.sync_copy(tod_hbm, tod_v)
    pltpu.sync_copy(dow_hbm, dow_v)
    plsc.subcore_barrier()

    iota = lax.iota(jnp.int32, 16)
    zeros = jnp.zeros((16,), jnp.int32)
    w_spl = [plsc.load_gather(wb_v, [zeros + (8 + c)]) for c in range(8)]
    b_spl = [plsc.load_gather(wb_v, [zeros + (16 + c)]) for c in range(8)]

    def x_in(li, x_v, sem):
        return pltpu.make_async_copy(x_hbm.at[wid, li], x_v, sem)

    def adp_out(li):
        return pltpu.make_async_copy(
            adp_sh.at[pl.ds(li * 8, 8)],
            out_hbm.at[wid, li, pl.ds(24, 8)], sem_ao)

    def sec_out(buf, li, c0, sem):
        return pltpu.make_async_copy(
            buf, out_hbm.at[wid, li, pl.ds(c0, 8)], sem)

    def proj_sec(x_v, li):
        @plsc.parallel_loop(0, GROUPS, unroll=2)
        def proj_body(g):
            x0 = x_v[0, pl.ds(g * 16, 16)]
            for c in range(8):
                buf_a[c, pl.ds(g * 16, 16)] = x0 * w_spl[c] + b_spl[c]
        sec_out(buf_a, li, 0, sem_a).start()

    def tod_sec(x_v, li):
        @plsc.parallel_loop(0, GROUPS, unroll=2)
        def tod_body(g):
            x3 = x_v[3, pl.ds(g * 16, 16)]
            ti = (x3 * float(STEP_PER_DAY)).astype(jnp.int32)
            for c in range(8):
                buf_b[c, pl.ds(g * 16, 16)] = plsc.load_gather(
                    tod_v, [zeros + c, ti])
        sec_out(buf_b, li, 8, sem_b).start()

    def dow_sec(x_v, li):
        @plsc.parallel_loop(0, GROUPS, unroll=2)
        def dow_body(g):
            x2 = x_v[2, pl.ds(g * 16, 16)]
            di = (x2 * float(DAY_PER_WEEK)).astype(jnp.int32)
            for c in range(8):
                buf_c[c, pl.ds(g * 16, 16)] = plsc.load_gather(
                    dow_v, [zeros + c, di])
        sec_out(buf_c, li, 16, sem_c).start()

    def phase(li, x_v, sem_x):
        x_in(li, x_v, sem_x).wait()

        @pl.when(li > 0)
        def _():
            sec_out(buf_a, li, 0, sem_a).wait()
            sec_out(buf_b, li, 8, sem_b).wait()
            sec_out(buf_c, li, 16, sem_c).wait()
            adp_out(li).wait()

        proj_sec(x_v, li)
        tod_sec(x_v, li)
        dow_sec(x_v, li)
        adp_out(li).start()

    def l_body(i, carry):
        l0 = 2 * i
        l1 = l0 + 1
        phase(l0, x_v0, sem_x0)
        x_in(l1, x_v1, sem_x1).start()
        phase(l1, x_v1, sem_x1)

        @pl.when(i < (L // 2) - 1)
        def _():
            x_in(l1 + 1, x_v0, sem_x0).start()
        return carry

    lax.fori_loop(0, L // 2, l_body, 0)

    sec_out(buf_a, L - 1, 0, sem_a).wait()
    sec_out(buf_b, L - 1, 8, sem_b).wait()
    sec_out(buf_c, L - 1, 16, sem_c).wait()
    adp_out(L - 1).wait()


@jax.jit
def _run(x_t, wb, tod_t, dow_t, adp_t):
    fn = pl.kernel(
        _sc_embed,
        out_type=jax.ShapeDtypeStruct((B, L, DW, N), jnp.float32),
        mesh=plsc.VectorSubcoreMesh(core_axis_name="c", subcore_axis_name="s"),
        compiler_params=pltpu.CompilerParams(needs_layout_passes=False),
        scratch_types=[
            pltpu.VMEM((C, N), jnp.float32),
            pltpu.VMEM((C, N), jnp.float32),
            pltpu.VMEM_SHARED(((L + 1) * 8, N), jnp.float32),
            pltpu.VMEM((8, N), jnp.float32),
            pltpu.VMEM((8, N), jnp.float32),
            pltpu.VMEM((8, N), jnp.float32),
            pltpu.VMEM((24,), jnp.float32),
            pltpu.VMEM((8, 24), jnp.float32),
            pltpu.VMEM((8, 8), jnp.float32),
            pltpu.SemaphoreType.DMA,
            pltpu.SemaphoreType.DMA,
            pltpu.SemaphoreType.DMA,
            pltpu.SemaphoreType.DMA,
            pltpu.SemaphoreType.DMA,
            pltpu.SemaphoreType.DMA,
            pltpu.SemaphoreType.DMA,
        ],
    )
    return fn(x_t, wb, tod_t, dow_t, adp_t)


def kernel(x, W_in, b_in, tod_table, dow_table, adp):
    x_t = x.transpose(0, 1, 3, 2)
    adp_t = adp.transpose(0, 2, 1).reshape(L * 8, N)
    wb = jnp.concatenate([jnp.zeros((8,), jnp.float32), W_in[0], b_in])
    tod_t = tod_table.T
    dow_t = jnp.zeros((8, 8), jnp.float32).at[:, :7].set(dow_table.T)
    out_t = _run(x_t, wb, tod_t, dow_t, adp_t)
    return out_t.transpose(0, 1, 3, 2)
